# Initial kernel scaffold; baseline (speedup 1.0000x reference)
import functools

import jax
import jax.numpy as jnp
from jax import lax
from jax.experimental import pallas as pl
from jax.experimental.pallas import tpu as pltpu
from jax.experimental.pallas import tpu_sc as plsc

B = 16384
L = 50
D = 32
LANES = 16

NC = 2
NS = 16
NW = NC * NS
BPW = B // NW
C = 16
PAIRS = C // 2
G = BPW // C
INV_L = 1.0 / L

_mesh = plsc.VectorSubcoreMesh(core_axis_name="c", subcore_axis_name="s")


@functools.partial(
    pl.kernel,
    mesh=_mesh,
    out_type=jax.ShapeDtypeStruct((B, D), jnp.float32),
    scratch_types=[
        pltpu.VMEM((PAIRS, 2 * L), jnp.int32),
        pltpu.VMEM((PAIRS, 2 * L, D), jnp.float32),
        pltpu.VMEM((C, D), jnp.float32),
        pltpu.SemaphoreType.DMA,
    ],
)
def _embed_mean(idx_hbm, table_hbm, out_hbm, idx_v, rows_v, out_v, sem):
    wid = lax.axis_index("s") * NC + lax.axis_index("c")
    pair_base = wid * (BPW // 2)
    batch_base = wid * BPW

    def group_body(g, _):
        pltpu.sync_copy(idx_hbm.at[pl.ds(pair_base + g * PAIRS, PAIRS)], idx_v)
        for j in range(PAIRS):
            pltpu.async_copy(table_hbm.at[idx_v.at[j]], rows_v.at[j], sem)
        for j in range(PAIRS):
            pltpu.make_async_copy(
                table_hbm.at[idx_v.at[j]], rows_v.at[j], sem).wait()

        def pair_body(p, _):
            for h in range(2):
                def r_body(r, accs):
                    a0, a1 = accs
                    for rr in range(5):
                        row = h * L + r * 5 + rr
                        a0 = a0 + rows_v[p, row, pl.ds(0, LANES)]
                        a1 = a1 + rows_v[p, row, pl.ds(LANES, LANES)]
                    return (a0, a1)

                a0, a1 = lax.fori_loop(
                    0, L // 5, r_body,
                    (jnp.zeros((LANES,), jnp.float32),
                     jnp.zeros((LANES,), jnp.float32)))
                out_v[2 * p + h, pl.ds(0, LANES)] = a0 * INV_L
                out_v[2 * p + h, pl.ds(LANES, LANES)] = a1 * INV_L
            return 0

        lax.fori_loop(0, PAIRS, pair_body, 0)
        pltpu.sync_copy(out_v, out_hbm.at[pl.ds(batch_base + g * C, C)])
        return 0

    lax.fori_loop(0, G, group_body, 0)


def kernel(inputs, embedding):
    idx = inputs.astype(jnp.int32).reshape(B // 2, 2 * L)
    return _embed_mean(idx, embedding)


# speedup vs baseline: 2.6670x; 2.6670x over previous
import functools

import jax
import jax.numpy as jnp
from jax import lax
from jax.experimental import pallas as pl
from jax.experimental.pallas import tpu as pltpu
from jax.experimental.pallas import tpu_sc as plsc

B = 16384
L = 50
D = 32
LANES = 16

NC = 2
NS = 16
NW = NC * NS
BPW = B // NW
C = 16
PAIRS = C // 2
G = BPW // C
INV_L = 1.0 / L

_mesh = plsc.VectorSubcoreMesh(core_axis_name="c", subcore_axis_name="s")


@functools.partial(
    pl.kernel,
    mesh=_mesh,
    compiler_params=pltpu.CompilerParams(use_tc_tiling_on_sc=False),
    out_type=jax.ShapeDtypeStruct((B, D), jnp.float32),
    scratch_types=[
        pltpu.VMEM((PAIRS, 2 * L), jnp.int32),
        pltpu.VMEM((PAIRS, 2 * L, D), jnp.float32),
        pltpu.VMEM((C, D), jnp.float32),
        pltpu.SemaphoreType.DMA,
    ],
)
def _embed_mean(idx_hbm, table_hbm, out_hbm, idx_v, rows_v, out_v, sem):
    wid = lax.axis_index("s") * NC + lax.axis_index("c")
    pair_base = wid * (BPW // 2)
    batch_base = wid * BPW

    def group_body(g, _):
        pltpu.sync_copy(idx_hbm.at[pl.ds(pair_base + g * PAIRS, PAIRS)], idx_v)
        for j in range(PAIRS):
            pltpu.async_copy(table_hbm.at[idx_v.at[j]], rows_v.at[j], sem)
        for j in range(PAIRS):
            pltpu.make_async_copy(
                table_hbm.at[idx_v.at[j]], rows_v.at[j], sem).wait()

        def pair_body(p, _):
            for h in range(2):
                def r_body(r, accs):
                    a0, a1 = accs
                    for rr in range(5):
                        row = h * L + r * 5 + rr
                        a0 = a0 + rows_v[p, row, pl.ds(0, LANES)]
                        a1 = a1 + rows_v[p, row, pl.ds(LANES, LANES)]
                    return (a0, a1)

                a0, a1 = lax.fori_loop(
                    0, L // 5, r_body,
                    (jnp.zeros((LANES,), jnp.float32),
                     jnp.zeros((LANES,), jnp.float32)))
                out_v[2 * p + h, pl.ds(0, LANES)] = a0 * INV_L
                out_v[2 * p + h, pl.ds(LANES, LANES)] = a1 * INV_L
            return 0

        lax.fori_loop(0, PAIRS, pair_body, 0)
        pltpu.sync_copy(out_v, out_hbm.at[pl.ds(batch_base + g * C, C)])
        return 0

    lax.fori_loop(0, G, group_body, 0)


def kernel(inputs, embedding):
    idx = inputs.astype(jnp.int32).reshape(B // 2, 2 * L)
    return _embed_mean(idx, embedding)


# speedup vs baseline: 3.7976x; 1.4239x over previous
import functools

import jax
import jax.numpy as jnp
from jax import lax
from jax.experimental import pallas as pl
from jax.experimental.pallas import tpu as pltpu
from jax.experimental.pallas import tpu_sc as plsc

B = 16384
L = 50
D = 32
V = 1000000
LANES = 16

NC = 2
NS = 16
NW = NC * NS

FULL_BLK = V // 128
QC = 1
NQ = FULL_BLK // QC
KMAX = (NQ + NW - 1) // NW
TAIL_W = FULL_BLK % NW
PACK_ROWS = (V + 3) // 4

BPW = B // NW
C = 16
PAIRS = C // 2
G = BPW // C
INV_L = 1.0 / L

_mesh = plsc.VectorSubcoreMesh(core_axis_name="c", subcore_axis_name="s")


@functools.partial(
    pl.kernel,
    mesh=_mesh,
    compiler_params=pltpu.CompilerParams(
        use_tc_tiling_on_sc=True, needs_layout_passes=False,
        disable_bounds_checks=True),
    out_type=jax.ShapeDtypeStruct((PACK_ROWS, 128), jnp.float32),
    scratch_types=[
        pltpu.VMEM((2, D, QC * 128), jnp.float32),
        pltpu.VMEM((2, QC * 32, 128), jnp.float32),
        pltpu.SemaphoreType.DMA,
        pltpu.SemaphoreType.DMA,
        pltpu.SemaphoreType.DMA,
        pltpu.SemaphoreType.DMA,
    ],
)
def _repack(embt_hbm, out_hbm, in_blks, out_blks, si0, si1, so0, so1):
    wid = lax.axis_index("s") * NC + lax.axis_index("c")
    sem_in = (si0, si1)
    sem_out = (so0, so1)
    iota0 = lax.iota(jnp.int32, 16)
    iota1 = iota0 + 16

    def in_copy(k, b):
        q = wid + k * NW
        return pltpu.make_async_copy(
            embt_hbm.at[:, pl.ds(q * (QC * 128), QC * 128)],
            in_blks.at[b], sem_in[b])

    def out_copy(k, b):
        q = wid + k * NW
        return pltpu.make_async_copy(
            out_blks.at[b], out_hbm.at[pl.ds(QC * 32 * q, QC * 32)],
            sem_out[b])

    perm = [(iota0 + t) & 15 for t in range(16)]
    pt4 = [p >> 2 for p in perm]
    lam = [[(p & 3) * 32 + iota0 + d0 for p in perm] for d0 in (0, 16)]

    def transpose_block(b, nh):
        @plsc.parallel_loop(0, nh, 1, unroll=2)
        def _body(h):
            c0 = h * 16
            r0 = h * 4
            for di, d0 in enumerate((0, 16)):
                rows = iota1 if d0 else iota0
                for t in range(16):
                    v = plsc.load_gather(in_blks.at[b], [rows, perm[t] + c0])
                    plsc.store_scatter(out_blks.at[b],
                                       [pt4[t] + r0, lam[di][t]], v)

    def step(k, b):
        pl.when(wid + (k + 1) * NW < NQ)(lambda: in_copy(k + 1, 1 - b).start())
        pl.when(wid + k * NW < NQ)(lambda: in_copy(k, b).wait())
        pl.when((k >= 2) & (wid + (k - 2) * NW < NQ))(
            lambda: out_copy(k - 2, b).wait())
        transpose_block(b, QC * 8)
        pl.when(wid + k * NW < NQ)(lambda: out_copy(k, b).start())

    pl.when(wid < NQ)(lambda: in_copy(0, 0).start())

    def pair_body(p, _):
        step(2 * p, 0)
        step(2 * p + 1, 1)
        return 0

    lax.fori_loop(0, (KMAX + 1) // 2, pair_body, 0)
    for k in (2 * ((KMAX + 1) // 2) - 2, 2 * ((KMAX + 1) // 2) - 1):
        pl.when(wid + k * NW < NQ)(lambda k=k: out_copy(k, k % 2).wait())

    if V % 128:
        @pl.when(wid == TAIL_W)
        def _():
            tail = V % 128
            trows = tail // 4
            for d in range(D):
                pltpu.sync_copy(embt_hbm.at[d, pl.ds(FULL_BLK * 128, tail)],
                                in_blks.at[0, d, pl.ds(0, tail)])
            transpose_block(0, tail // 16)
            pltpu.sync_copy(out_blks.at[0, pl.ds(0, trows)],
                            out_hbm.at[pl.ds(32 * FULL_BLK, trows)])


@functools.partial(
    pl.kernel,
    mesh=_mesh,
    compiler_params=pltpu.CompilerParams(use_tc_tiling_on_sc=False),
    out_type=jax.ShapeDtypeStruct((B, D), jnp.float32),
    scratch_types=[
        pltpu.VMEM((PAIRS, 2 * L), jnp.int32),
        pltpu.VMEM((PAIRS, 2 * L, D), jnp.float32),
        pltpu.VMEM((C, D), jnp.float32),
        pltpu.SemaphoreType.DMA,
    ],
)
def _embed_mean(idx_hbm, table_hbm, out_hbm, idx_v, rows_v, out_v, sem):
    wid = lax.axis_index("s") * NC + lax.axis_index("c")
    pair_base = wid * (BPW // 2)
    batch_base = wid * BPW

    def group_body(g, _):
        pltpu.sync_copy(idx_hbm.at[pl.ds(pair_base + g * PAIRS, PAIRS)], idx_v)
        for j in range(PAIRS):
            pltpu.async_copy(table_hbm.at[idx_v.at[j]], rows_v.at[j], sem)
        for j in range(PAIRS):
            pltpu.make_async_copy(
                table_hbm.at[idx_v.at[j]], rows_v.at[j], sem).wait()

        def pair_body(p, _):
            for h in range(2):
                def r_body(r, accs):
                    a0, a1 = accs
                    for rr in range(5):
                        row = h * L + r * 5 + rr
                        a0 = a0 + rows_v[p, row, pl.ds(0, LANES)]
                        a1 = a1 + rows_v[p, row, pl.ds(LANES, LANES)]
                    return (a0, a1)

                a0, a1 = lax.fori_loop(
                    0, L // 5, r_body,
                    (jnp.zeros((LANES,), jnp.float32),
                     jnp.zeros((LANES,), jnp.float32)))
                out_v[2 * p + h, pl.ds(0, LANES)] = a0 * INV_L
                out_v[2 * p + h, pl.ds(LANES, LANES)] = a1 * INV_L
            return 0

        lax.fori_loop(0, PAIRS, pair_body, 0)
        pltpu.sync_copy(out_v, out_hbm.at[pl.ds(batch_base + g * C, C)])
        return 0

    lax.fori_loop(0, G, group_body, 0)


def kernel(inputs, embedding):
    idx = inputs.astype(jnp.int32).reshape(B // 2, 2 * L)
    packed = _repack(embedding.T)
    return _embed_mean(idx, packed.reshape(V, D))


# speedup vs baseline: 5.1373x; 1.3528x over previous
import functools

import jax
import jax.numpy as jnp
from jax import lax
from jax.experimental import pallas as pl
from jax.experimental.pallas import tpu as pltpu
from jax.experimental.pallas import tpu_sc as plsc

B = 16384
L = 50
D = 32
V = 1000000
LANES = 16

NC = 2
NS = 16
NW = NC * NS

FULL_BLK = V // 128
QC = 1
NQ = FULL_BLK // QC
KMAX = (NQ + NW - 1) // NW
TAIL_W = FULL_BLK % NW
PACK_ROWS = (V + 3) // 4

BPW = B // NW
C = 16
PAIRS = C // 2
G = BPW // C
INV_L = 1.0 / L

_mesh = plsc.VectorSubcoreMesh(core_axis_name="c", subcore_axis_name="s")


@functools.partial(
    pl.kernel,
    mesh=_mesh,
    compiler_params=pltpu.CompilerParams(
        use_tc_tiling_on_sc=True, needs_layout_passes=False,
        disable_bounds_checks=True),
    out_type=jax.ShapeDtypeStruct((PACK_ROWS, 128), jnp.float32),
    scratch_types=[
        pltpu.VMEM((2, D, QC * 128), jnp.float32),
        pltpu.VMEM((2, QC * 32, 128), jnp.float32),
        pltpu.SemaphoreType.DMA,
        pltpu.SemaphoreType.DMA,
        pltpu.SemaphoreType.DMA,
        pltpu.SemaphoreType.DMA,
    ],
)
def _repack(embt_hbm, out_hbm, in_blks, out_blks, si0, si1, so0, so1):
    wid = lax.axis_index("s") * NC + lax.axis_index("c")
    sem_in = (si0, si1)
    sem_out = (so0, so1)
    iota0 = lax.iota(jnp.int32, 16)
    iota1 = iota0 + 16

    def in_copy(k, b):
        q = wid + k * NW
        return pltpu.make_async_copy(
            embt_hbm.at[:, pl.ds(q * (QC * 128), QC * 128)],
            in_blks.at[b], sem_in[b])

    def out_copy(k, b):
        q = wid + k * NW
        return pltpu.make_async_copy(
            out_blks.at[b], out_hbm.at[pl.ds(QC * 32 * q, QC * 32)],
            sem_out[b])

    perm = [(iota0 + t) & 15 for t in range(16)]
    pt4 = [p >> 2 for p in perm]
    lam = [[(p & 3) * 32 + iota0 + d0 for p in perm] for d0 in (0, 16)]

    def transpose_block(b, nh):
        @plsc.parallel_loop(0, nh, 1, unroll=4)
        def _body(h):
            c0 = h * 16
            r0 = h * 4
            for di, d0 in enumerate((0, 16)):
                rows = iota1 if d0 else iota0
                for t in range(16):
                    v = plsc.load_gather(in_blks.at[b], [rows, perm[t] + c0])
                    plsc.store_scatter(out_blks.at[b],
                                       [pt4[t] + r0, lam[di][t]], v)

    def step(k, b):
        pl.when(wid + (k + 1) * NW < NQ)(lambda: in_copy(k + 1, 1 - b).start())
        pl.when(wid + k * NW < NQ)(lambda: in_copy(k, b).wait())
        pl.when((k >= 2) & (wid + (k - 2) * NW < NQ))(
            lambda: out_copy(k - 2, b).wait())
        transpose_block(b, QC * 8)
        pl.when(wid + k * NW < NQ)(lambda: out_copy(k, b).start())

    pl.when(wid < NQ)(lambda: in_copy(0, 0).start())

    def pair_body(p, _):
        step(2 * p, 0)
        step(2 * p + 1, 1)
        return 0

    lax.fori_loop(0, (KMAX + 1) // 2, pair_body, 0)
    for k in (2 * ((KMAX + 1) // 2) - 2, 2 * ((KMAX + 1) // 2) - 1):
        pl.when(wid + k * NW < NQ)(lambda k=k: out_copy(k, k % 2).wait())

    if V % 128:
        @pl.when(wid == TAIL_W)
        def _():
            tail = V % 128
            trows = tail // 4
            for d in range(D):
                pltpu.sync_copy(embt_hbm.at[d, pl.ds(FULL_BLK * 128, tail)],
                                in_blks.at[0, d, pl.ds(0, tail)])
            transpose_block(0, tail // 16)
            pltpu.sync_copy(out_blks.at[0, pl.ds(0, trows)],
                            out_hbm.at[pl.ds(32 * FULL_BLK, trows)])


@functools.partial(
    pl.kernel,
    mesh=_mesh,
    compiler_params=pltpu.CompilerParams(use_tc_tiling_on_sc=False),
    out_type=jax.ShapeDtypeStruct((B, D), jnp.float32),
    scratch_types=[
        pltpu.VMEM((PAIRS, 2 * L), jnp.int32),
        pltpu.VMEM((PAIRS, 2 * L, D), jnp.float32),
        pltpu.VMEM((C, D), jnp.float32),
        pltpu.SemaphoreType.DMA,
    ],
)
def _embed_mean(idx_hbm, table_hbm, out_hbm, idx_v, rows_v, out_v, sem):
    wid = lax.axis_index("s") * NC + lax.axis_index("c")
    pair_base = wid * (BPW // 2)
    batch_base = wid * BPW

    def group_body(g, _):
        pltpu.sync_copy(idx_hbm.at[pl.ds(pair_base + g * PAIRS, PAIRS)], idx_v)
        for j in range(PAIRS):
            pltpu.async_copy(table_hbm.at[idx_v.at[j]], rows_v.at[j], sem)
        for j in range(PAIRS):
            pltpu.make_async_copy(
                table_hbm.at[idx_v.at[j]], rows_v.at[j], sem).wait()

        def pair_body(p, _):
            for h in range(2):
                def r_body(r, accs):
                    a0, a1 = accs
                    for rr in range(5):
                        row = h * L + r * 5 + rr
                        a0 = a0 + rows_v[p, row, pl.ds(0, LANES)]
                        a1 = a1 + rows_v[p, row, pl.ds(LANES, LANES)]
                    return (a0, a1)

                a0, a1 = lax.fori_loop(
                    0, L // 5, r_body,
                    (jnp.zeros((LANES,), jnp.float32),
                     jnp.zeros((LANES,), jnp.float32)))
                out_v[2 * p + h, pl.ds(0, LANES)] = a0 * INV_L
                out_v[2 * p + h, pl.ds(LANES, LANES)] = a1 * INV_L
            return 0

        lax.fori_loop(0, PAIRS, pair_body, 0)
        pltpu.sync_copy(out_v, out_hbm.at[pl.ds(batch_base + g * C, C)])
        return 0

    lax.fori_loop(0, G, group_body, 0)


def kernel(inputs, embedding):
    idx = inputs.astype(jnp.int32).reshape(B // 2, 2 * L)
    packed = _repack(embedding.T)
    return _embed_mean(idx, packed.reshape(V, D))
